# Initial kernel scaffold; baseline (speedup 1.0000x reference)
import jax
import jax.numpy as jnp
from jax.experimental import pallas as pl


def kernel(x, targets, table):
    raise NotImplementedError("write your pallas kernel here")


# speedup vs baseline: 1.8372x; 1.8372x over previous
import functools

import jax
import jax.numpy as jnp
from jax import lax
from jax.experimental import pallas as pl
from jax.experimental.pallas import tpu as pltpu
from jax.experimental.pallas import tpu_sc as plsc

V = 8192
B = 16
T = 1024
N = B * T

_NC = 2
_NS = 16
_L = 16
_NW = _NC * _NS
_PER_W = N // _NW
_CH = 8
_NCHUNK = _PER_W // _CH


def _mesh():
    return plsc.VectorSubcoreMesh(core_axis_name="c", subcore_axis_name="s")


def _sc_gather(table, xf):
    @functools.partial(
        pl.kernel,
        mesh=_mesh(),
        out_type=jax.ShapeDtypeStruct((N, V), jnp.float32),
        scratch_types=[
            pltpu.VMEM((_PER_W,), jnp.int32),
            pltpu.VMEM((_CH, V), jnp.float32),
            pltpu.SemaphoreType.DMA,
        ],
    )
    def k(table_hbm, x_hbm, out_hbm, xv, rows, sem):
        wid = lax.axis_index("s") * _NC + lax.axis_index("c")
        base = wid * _PER_W
        pltpu.sync_copy(x_hbm.at[pl.ds(base, _PER_W)], xv)

        def chunk(j, carry):
            off = j * _CH
            pltpu.async_copy(
                table_hbm.at[xv.at[pl.ds(off, _CH)]], rows, sem
            ).wait()
            pltpu.sync_copy(rows, out_hbm.at[pl.ds(base + off, _CH)])
            return carry

        lax.fori_loop(0, _NCHUNK, chunk, 0)

    return k(table, xf)


_LSE_ROWS = 256


def _lse_body(tbl_ref, out_ref):
    blk = tbl_ref[...]
    m = jnp.max(blk, axis=1, keepdims=True)
    s = jnp.sum(jnp.exp(blk - m), axis=1, keepdims=True)
    out_ref[...] = jnp.log(s) + m


def _table_lse(table):
    return pl.pallas_call(
        _lse_body,
        grid=(V // _LSE_ROWS,),
        in_specs=[pl.BlockSpec((_LSE_ROWS, V), lambda i: (i, 0))],
        out_specs=pl.BlockSpec((_LSE_ROWS, 1), lambda i: (i, 0)),
        out_shape=jax.ShapeDtypeStruct((V, 1), jnp.float32),
    )(table)


def _sc_combine(table_flat, xf, tf, lse):
    @functools.partial(
        pl.kernel,
        mesh=_mesh(),
        out_type=jax.ShapeDtypeStruct((_NW, _L), jnp.float32),
        scratch_types=[
            pltpu.VMEM((_PER_W,), jnp.int32),
            pltpu.VMEM((_PER_W,), jnp.int32),
            pltpu.VMEM((_PER_W,), jnp.int32),
            pltpu.VMEM((_PER_W,), jnp.float32),
            pltpu.VMEM((_PER_W,), jnp.float32),
            pltpu.VMEM((_L,), jnp.float32),
            pltpu.SemaphoreType.DMA,
        ],
    )
    def k(tbl_hbm, x_hbm, t_hbm, lse_hbm, out_hbm, xv, tv, fv, lv, gv, accv,
          sem):
        wid = lax.axis_index("s") * _NC + lax.axis_index("c")
        base = wid * _PER_W
        pltpu.sync_copy(x_hbm.at[pl.ds(base, _PER_W)], xv)
        pltpu.sync_copy(t_hbm.at[pl.ds(base, _PER_W)], tv)

        def mkflat(i, c):
            sl = pl.ds(i * _L, _L)
            fv[sl] = xv[sl] * V + tv[sl]
            return c

        lax.fori_loop(0, _PER_W // _L, mkflat, 0)

        def gchunk(j, c):
            sl = pl.ds(j * 128, 128)
            pltpu.async_copy(lse_hbm.at[xv.at[sl]], lv.at[sl], sem).wait()
            pltpu.async_copy(tbl_hbm.at[fv.at[sl]], gv.at[sl], sem).wait()
            return c

        lax.fori_loop(0, _PER_W // 128, gchunk, 0)

        def red(i, acc):
            sl = pl.ds(i * _L, _L)
            return acc + (lv[sl] - gv[sl])

        acc = lax.fori_loop(0, _PER_W // _L, red, jnp.zeros((_L,), jnp.float32))
        accv[...] = acc
        pltpu.sync_copy(accv, out_hbm.at[wid])

    return k(table_flat, xf, tf, lse)


def _loss_body(p_ref, o_ref):
    o_ref[...] = (jnp.sum(p_ref[...]) / N).reshape(1, 1)


def _loss_sum(partials):
    return pl.pallas_call(
        _loss_body,
        out_shape=jax.ShapeDtypeStruct((1, 1), jnp.float32),
    )(partials)


def kernel(x, targets, table):
    xf = x.reshape(-1)
    tf = targets.reshape(-1)
    logits2d = _sc_gather(table, xf)
    lse = _table_lse(table).reshape(-1)
    partials = _sc_combine(table.reshape(-1), xf, tf, lse)
    loss = _loss_sum(partials)[0, 0]
    return logits2d.reshape(B, T, V), loss
